# baseline (speedup 1.0000x reference)
import functools

import jax
import jax.numpy as jnp
from jax import lax
from jax.experimental import pallas as pl
from jax.experimental.pallas import tpu as pltpu
from jax.experimental.pallas import tpu_sc as plsc

BATCH = 16384
EMBED_DIM = 32

_INFO = plsc.get_sparse_core_info()
_NC = _INFO.num_cores
_NS = _INFO.num_subcores
_NW = _NC * _NS
_BPW = BATCH // _NW


@functools.partial(
    pl.kernel,
    mesh=plsc.VectorSubcoreMesh(core_axis_name="c", subcore_axis_name="s"),
    out_type=(
        jax.ShapeDtypeStruct((BATCH, EMBED_DIM), jnp.float32),
        jax.ShapeDtypeStruct((BATCH, EMBED_DIM), jnp.float32),
    ),
    scratch_types=[
        pltpu.VMEM((_BPW,), jnp.int32),
        pltpu.VMEM((_BPW,), jnp.int32),
        pltpu.VMEM((_BPW, EMBED_DIM), jnp.float32),
        pltpu.VMEM((_BPW, EMBED_DIM), jnp.float32),
        pltpu.SemaphoreType.DMA,
        pltpu.SemaphoreType.DMA,
    ],
    compiler_params=pltpu.CompilerParams(use_tc_tiling_on_sc=False),
)
def _sc_gather(uid_hbm, iid_hbm, utab_hbm, itab_hbm, uout_hbm, iout_hbm,
               uidx_v, iidx_v, urows_v, irows_v, sem_u, sem_i):
    wid = lax.axis_index("s") * _NC + lax.axis_index("c")
    base = wid * _BPW
    pltpu.sync_copy(uid_hbm.at[pl.ds(base, _BPW)], uidx_v)
    pltpu.sync_copy(iid_hbm.at[pl.ds(base, _BPW)], iidx_v)
    cu = pltpu.async_copy(utab_hbm.at[uidx_v], urows_v, sem_u)
    ci = pltpu.async_copy(itab_hbm.at[iidx_v], irows_v, sem_i)
    cu.wait()
    ci.wait()
    pltpu.sync_copy(urows_v, uout_hbm.at[pl.ds(base, _BPW)])
    pltpu.sync_copy(irows_v, iout_hbm.at[pl.ds(base, _BPW)])


_BB = 2048


def _mlp_body(xu_ref, xi_ref, w1u_ref, w1i_ref, b1_ref, w2_ref, b2_ref, o_ref):
    h = (
        jnp.dot(xu_ref[...], w1u_ref[...], preferred_element_type=jnp.float32)
        + jnp.dot(xi_ref[...], w1i_ref[...], preferred_element_type=jnp.float32)
        + b1_ref[...]
    )
    h = jnp.maximum(h, 0.0)
    o_ref[...] = (
        jnp.dot(h, w2_ref[...], preferred_element_type=jnp.float32) + b2_ref[...]
    )


def _tc_mlp(xu, xi, w1u, w1i, b1, w2, b2):
    grid = (BATCH // _BB,)
    return pl.pallas_call(
        _mlp_body,
        grid=grid,
        in_specs=[
            pl.BlockSpec((_BB, EMBED_DIM), lambda i: (i, 0)),
            pl.BlockSpec((_BB, EMBED_DIM), lambda i: (i, 0)),
            pl.BlockSpec((EMBED_DIM, 64), lambda i: (0, 0)),
            pl.BlockSpec((EMBED_DIM, 64), lambda i: (0, 0)),
            pl.BlockSpec((1, 64), lambda i: (0, 0)),
            pl.BlockSpec((64, 11), lambda i: (0, 0)),
            pl.BlockSpec((1, 11), lambda i: (0, 0)),
        ],
        out_specs=pl.BlockSpec((_BB, 11), lambda i: (i, 0)),
        out_shape=jax.ShapeDtypeStruct((BATCH, 11), jnp.float32),
    )(xu, xi, w1u, w1i, b1, w2, b2)


def kernel(user_ids, item_ids, user_table, item_table, W1, b1, W2, b2):
    uvec, ivec = _sc_gather(
        user_ids.astype(jnp.int32), item_ids.astype(jnp.int32),
        user_table, item_table,
    )
    return _tc_mlp(
        uvec, ivec,
        W1[:EMBED_DIM], W1[EMBED_DIM:],
        b1.reshape(1, 64), W2, b2.reshape(1, 11),
    )


# speedup vs baseline: 1.5234x; 1.5234x over previous
import functools

import jax
import jax.numpy as jnp
from jax import lax
from jax.experimental import pallas as pl
from jax.experimental.pallas import tpu as pltpu
from jax.experimental.pallas import tpu_sc as plsc

BATCH = 16384
EMBED_DIM = 32
NROWS = 1000000
PACK = 4
GSIZE = 1 << 18
PROWS = GSIZE
PWIDTH = EMBED_DIM * PACK

_INFO = plsc.get_sparse_core_info()
_NC = _INFO.num_cores
_NS = _INFO.num_subcores
_NW = _NC * _NS
_BPW = BATCH // _NW
_CHUNK = 128
_NCHUNK = _BPW // _CHUNK


_UBLK = 2048


def _repack_body(t0_ref, t1_ref, t2_ref, t3_ref, o_ref):
    for g, t_ref in enumerate((t0_ref, t1_ref, t2_ref, t3_ref)):
        o_ref[:, g * EMBED_DIM:(g + 1) * EMBED_DIM] = t_ref[...].T


def _tc_repack(tab_t):
    grid = (GSIZE // _UBLK,)
    gstride = GSIZE // _UBLK
    last_blk = NROWS // _UBLK

    def in_map(g):
        return lambda i: (0, jnp.minimum(g * gstride + i, last_blk))

    return pl.pallas_call(
        _repack_body,
        grid=grid,
        in_specs=[pl.BlockSpec((EMBED_DIM, _UBLK), in_map(g))
                  for g in range(PACK)],
        out_specs=pl.BlockSpec((_UBLK, PWIDTH), lambda i: (i, 0)),
        out_shape=jax.ShapeDtypeStruct((PROWS, PWIDTH), jnp.float32),
    )(tab_t, tab_t, tab_t, tab_t)


@functools.partial(
    pl.kernel,
    mesh=plsc.VectorSubcoreMesh(core_axis_name="c", subcore_axis_name="s"),
    out_type=jax.ShapeDtypeStruct((BATCH, PWIDTH), jnp.float32),
    scratch_types=[
        pltpu.VMEM((_BPW,), jnp.int32),
        pltpu.VMEM((_BPW, PWIDTH), jnp.float32),
        pltpu.SemaphoreType.DMA,
    ],
)
def _sc_gather(idx_hbm, ptab_hbm, out_hbm, idx_v, rows_v, sem):
    wid = lax.axis_index("s") * _NC + lax.axis_index("c")
    base = wid * _BPW
    pltpu.sync_copy(idx_hbm.at[pl.ds(base, _BPW)], idx_v)
    pltpu.async_copy(ptab_hbm.at[idx_v], rows_v, sem).wait()
    pltpu.sync_copy(rows_v, out_hbm.at[pl.ds(base, _BPW)])


_BB = 2048


def _mlp_body(xu_ref, xi_ref, idu_ref, idi_ref, w1u_ref, w1i_ref, b1_ref,
              w2_ref, b2_ref, o_ref):
    def select32(x128, ids):
        g = ids >> 18
        out = jnp.zeros((_BB, EMBED_DIM), jnp.float32)
        for k in range(PACK):
            out = out + jnp.where(
                g == k, x128[:, k * EMBED_DIM:(k + 1) * EMBED_DIM], 0.0)
        return out

    xu = select32(xu_ref[...], idu_ref[...])
    xi = select32(xi_ref[...], idi_ref[...])
    h = (
        jnp.dot(xu, w1u_ref[...], preferred_element_type=jnp.float32)
        + jnp.dot(xi, w1i_ref[...], preferred_element_type=jnp.float32)
        + b1_ref[...]
    )
    h = jnp.maximum(h, 0.0)
    o_ref[...] = (
        jnp.dot(h, w2_ref[...], preferred_element_type=jnp.float32) + b2_ref[...]
    )


def _tc_mlp(xu, xi, idu, idi, w1u, w1i, b1, w2, b2):
    grid = (BATCH // _BB,)
    return pl.pallas_call(
        _mlp_body,
        grid=grid,
        in_specs=[
            pl.BlockSpec((_BB, PWIDTH), lambda i: (i, 0)),
            pl.BlockSpec((_BB, PWIDTH), lambda i: (i, 0)),
            pl.BlockSpec((_BB, 1), lambda i: (i, 0)),
            pl.BlockSpec((_BB, 1), lambda i: (i, 0)),
            pl.BlockSpec((EMBED_DIM, 64), lambda i: (0, 0)),
            pl.BlockSpec((EMBED_DIM, 64), lambda i: (0, 0)),
            pl.BlockSpec((1, 64), lambda i: (0, 0)),
            pl.BlockSpec((64, 11), lambda i: (0, 0)),
            pl.BlockSpec((1, 11), lambda i: (0, 0)),
        ],
        out_specs=pl.BlockSpec((_BB, 11), lambda i: (i, 0)),
        out_shape=jax.ShapeDtypeStruct((BATCH, 11), jnp.float32),
    )(xu, xi, idu, idi, w1u, w1i, b1, w2, b2)


def kernel(user_ids, item_ids, user_table, item_table, W1, b1, W2, b2):
    uid = user_ids.astype(jnp.int32)
    iid = item_ids.astype(jnp.int32)
    uptab = _tc_repack(user_table.T)
    urows = _sc_gather(uid & (GSIZE - 1), uptab)
    iptab = _tc_repack(item_table.T)
    irows = _sc_gather(iid & (GSIZE - 1), iptab)
    return _tc_mlp(
        urows, irows,
        uid.reshape(BATCH, 1), iid.reshape(BATCH, 1),
        W1[:EMBED_DIM], W1[EMBED_DIM:],
        b1.reshape(1, 64), W2, b2.reshape(1, 11),
    )


# speedup vs baseline: 2.4377x; 1.6002x over previous
import functools

import jax
import jax.numpy as jnp
from jax import lax
from jax.experimental import pallas as pl
from jax.experimental.pallas import tpu as pltpu
from jax.experimental.pallas import tpu_sc as plsc

BATCH = 16384
EMBED_DIM = 32
NROWS = 1000000
PACK = 4
GSIZE = 1 << 18
PROWS = GSIZE
PWIDTH = EMBED_DIM * PACK

_INFO = plsc.get_sparse_core_info()
_NC = _INFO.num_cores
_NS = _INFO.num_subcores
_NW = _NC * _NS
_BPW = BATCH // _NW
_CHUNK = 128
_NCHUNK = _BPW // _CHUNK


_UBLK = 4096

_CONTRACT00 = (((0,), (0,)), ((), ()))


def _repack_body(t0_ref, t1_ref, t2_ref, t3_ref, s0_ref, s1_ref, s2_ref,
                 s3_ref, o_ref):
    acc = jax.lax.dot_general(t0_ref[...], s0_ref[...], _CONTRACT00,
                              preferred_element_type=jnp.float32)
    for t_ref, s_ref in ((t1_ref, s1_ref), (t2_ref, s2_ref), (t3_ref, s3_ref)):
        acc = acc + jax.lax.dot_general(t_ref[...], s_ref[...], _CONTRACT00,
                                        preferred_element_type=jnp.float32)
    o_ref[...] = acc


def _tc_repack(tab_t, sel):
    grid = (GSIZE // _UBLK,)
    gstride = GSIZE // _UBLK
    last_blk = NROWS // _UBLK

    def in_map(g):
        return lambda i: (0, jnp.minimum(g * gstride + i, last_blk))

    return pl.pallas_call(
        _repack_body,
        grid=grid,
        in_specs=(
            [pl.BlockSpec((EMBED_DIM, _UBLK), in_map(g)) for g in range(PACK)]
            + [pl.BlockSpec((EMBED_DIM, PWIDTH), lambda i: (0, 0))] * PACK
        ),
        out_specs=pl.BlockSpec((_UBLK, PWIDTH), lambda i: (i, 0)),
        out_shape=jax.ShapeDtypeStruct((PROWS, PWIDTH), jnp.float32),
    )(tab_t, tab_t, tab_t, tab_t, sel[0], sel[1], sel[2], sel[3])


@functools.partial(
    pl.kernel,
    mesh=plsc.VectorSubcoreMesh(core_axis_name="c", subcore_axis_name="s"),
    out_type=jax.ShapeDtypeStruct((BATCH, PWIDTH), jnp.float32),
    scratch_types=[
        pltpu.VMEM((_BPW,), jnp.int32),
        pltpu.VMEM((_BPW, PWIDTH), jnp.float32),
        pltpu.SemaphoreType.DMA,
    ],
)
def _sc_gather(idx_hbm, ptab_hbm, out_hbm, idx_v, rows_v, sem):
    wid = lax.axis_index("s") * _NC + lax.axis_index("c")
    base = wid * _BPW
    pltpu.sync_copy(idx_hbm.at[pl.ds(base, _BPW)], idx_v)
    pltpu.async_copy(ptab_hbm.at[idx_v], rows_v, sem).wait()
    pltpu.sync_copy(rows_v, out_hbm.at[pl.ds(base, _BPW)])


_BB = 2048


def _mlp_body(xu_ref, xi_ref, idu_ref, idi_ref, w1u_ref, w1i_ref, b1_ref,
              w2_ref, b2_ref, o_ref):
    lane_g = jax.lax.broadcasted_iota(jnp.int32, (_BB, PWIDTH), 1) >> 5

    def masked(x128, ids):
        return jnp.where(lane_g == (ids >> 18), x128, 0.0)

    xu = masked(xu_ref[...], idu_ref[...])
    xi = masked(xi_ref[...], idi_ref[...])
    h = (
        jnp.dot(xu, w1u_ref[...], preferred_element_type=jnp.float32)
        + jnp.dot(xi, w1i_ref[...], preferred_element_type=jnp.float32)
        + b1_ref[...]
    )
    h = jnp.maximum(h, 0.0)
    o_ref[...] = (
        jnp.dot(h, w2_ref[...], preferred_element_type=jnp.float32) + b2_ref[...]
    )


def _tc_mlp(xu, xi, idu, idi, w1u, w1i, b1, w2, b2):
    grid = (BATCH // _BB,)
    return pl.pallas_call(
        _mlp_body,
        grid=grid,
        in_specs=[
            pl.BlockSpec((_BB, PWIDTH), lambda i: (i, 0)),
            pl.BlockSpec((_BB, PWIDTH), lambda i: (i, 0)),
            pl.BlockSpec((_BB, 1), lambda i: (i, 0)),
            pl.BlockSpec((_BB, 1), lambda i: (i, 0)),
            pl.BlockSpec((PWIDTH, 64), lambda i: (0, 0)),
            pl.BlockSpec((PWIDTH, 64), lambda i: (0, 0)),
            pl.BlockSpec((1, 64), lambda i: (0, 0)),
            pl.BlockSpec((64, 11), lambda i: (0, 0)),
            pl.BlockSpec((1, 11), lambda i: (0, 0)),
        ],
        out_specs=pl.BlockSpec((_BB, 11), lambda i: (i, 0)),
        out_shape=jax.ShapeDtypeStruct((BATCH, 11), jnp.float32),
    )(xu, xi, idu, idi, w1u, w1i, b1, w2, b2)


def kernel(user_ids, item_ids, user_table, item_table, W1, b1, W2, b2):
    uid = user_ids.astype(jnp.int32)
    iid = item_ids.astype(jnp.int32)
    eye = jnp.eye(PWIDTH, dtype=jnp.float32)
    sel = [eye[g * EMBED_DIM:(g + 1) * EMBED_DIM] for g in range(PACK)]
    uptab = _tc_repack(user_table.T, sel)
    urows = _sc_gather(uid & (GSIZE - 1), uptab)
    iptab = _tc_repack(item_table.T, sel)
    irows = _sc_gather(iid & (GSIZE - 1), iptab)
    return _tc_mlp(
        urows, irows,
        uid.reshape(BATCH, 1), iid.reshape(BATCH, 1),
        jnp.tile(W1[:EMBED_DIM], (PACK, 1)), jnp.tile(W1[EMBED_DIM:], (PACK, 1)),
        b1.reshape(1, 64), W2, b2.reshape(1, 11),
    )


# speedup vs baseline: 3.5210x; 1.4444x over previous
import functools

import jax
import jax.numpy as jnp
from jax import lax
from jax.experimental import pallas as pl
from jax.experimental.pallas import tpu as pltpu
from jax.experimental.pallas import tpu_sc as plsc

BATCH = 16384
EMBED_DIM = 32
NROWS = 1000000
PACK = 4
GSIZE = 1 << 18
PROWS = GSIZE
PWIDTH = EMBED_DIM * PACK

_INFO = plsc.get_sparse_core_info()
_NC = _INFO.num_cores
_NS = _INFO.num_subcores
_NW = _NC * _NS
_BPW = BATCH // _NW
_CHUNK = 128
_NCHUNK = _BPW // _CHUNK


_UBLK = 4096

_CONTRACT00 = (((0,), (0,)), ((), ()))


def _repack_body(t0_ref, t1_ref, t2_ref, t3_ref, eye_ref, o_ref):
    x = jnp.concatenate(
        [t0_ref[...], t1_ref[...], t2_ref[...], t3_ref[...]], axis=0)
    o_ref[...] = jax.lax.dot_general(x, eye_ref[...], _CONTRACT00,
                                     preferred_element_type=jnp.float32)


def _tc_repack(tab_t, eye):
    grid = (GSIZE // _UBLK,)
    gstride = GSIZE // _UBLK
    last_blk = NROWS // _UBLK

    def in_map(g):
        return lambda i: (0, jnp.minimum(g * gstride + i, last_blk))

    return pl.pallas_call(
        _repack_body,
        grid=grid,
        in_specs=(
            [pl.BlockSpec((EMBED_DIM, _UBLK), in_map(g)) for g in range(PACK)]
            + [pl.BlockSpec((PWIDTH, PWIDTH), lambda i: (0, 0))]
        ),
        out_specs=pl.BlockSpec((_UBLK, PWIDTH), lambda i: (i, 0)),
        out_shape=jax.ShapeDtypeStruct((PROWS, PWIDTH), jnp.float32),
    )(tab_t, tab_t, tab_t, tab_t, eye)


@functools.partial(
    pl.kernel,
    mesh=plsc.VectorSubcoreMesh(core_axis_name="c", subcore_axis_name="s"),
    out_type=jax.ShapeDtypeStruct((BATCH, PWIDTH), jnp.float32),
    scratch_types=[
        pltpu.VMEM((_BPW,), jnp.int32),
        pltpu.VMEM((_BPW, PWIDTH), jnp.float32),
        pltpu.SemaphoreType.DMA,
    ],
)
def _sc_gather(idx_hbm, ptab_hbm, out_hbm, idx_v, rows_v, sem):
    wid = lax.axis_index("s") * _NC + lax.axis_index("c")
    base = wid * _BPW
    pltpu.sync_copy(idx_hbm.at[pl.ds(base, _BPW)], idx_v)
    pltpu.async_copy(ptab_hbm.at[idx_v], rows_v, sem).wait()
    pltpu.sync_copy(rows_v, out_hbm.at[pl.ds(base, _BPW)])


_BB = 2048


def _mlp_body(xu_ref, xi_ref, idu_ref, idi_ref, w1u_ref, w1i_ref, b1_ref,
              w2_ref, b2_ref, o_ref):
    lane_g = jax.lax.broadcasted_iota(jnp.int32, (_BB, PWIDTH), 1) >> 5

    def masked(x128, ids):
        return jnp.where(lane_g == (ids >> 18), x128, 0.0)

    xu = masked(xu_ref[...], idu_ref[...])
    xi = masked(xi_ref[...], idi_ref[...])
    h = (
        jnp.dot(xu, w1u_ref[...], preferred_element_type=jnp.float32)
        + jnp.dot(xi, w1i_ref[...], preferred_element_type=jnp.float32)
        + b1_ref[...]
    )
    h = jnp.maximum(h, 0.0)
    o_ref[...] = (
        jnp.dot(h, w2_ref[...], preferred_element_type=jnp.float32) + b2_ref[...]
    )


def _tc_mlp(xu, xi, idu, idi, w1u, w1i, b1, w2, b2):
    grid = (BATCH // _BB,)
    return pl.pallas_call(
        _mlp_body,
        grid=grid,
        in_specs=[
            pl.BlockSpec((_BB, PWIDTH), lambda i: (i, 0)),
            pl.BlockSpec((_BB, PWIDTH), lambda i: (i, 0)),
            pl.BlockSpec((_BB, 1), lambda i: (i, 0)),
            pl.BlockSpec((_BB, 1), lambda i: (i, 0)),
            pl.BlockSpec((PWIDTH, 64), lambda i: (0, 0)),
            pl.BlockSpec((PWIDTH, 64), lambda i: (0, 0)),
            pl.BlockSpec((1, 64), lambda i: (0, 0)),
            pl.BlockSpec((64, 11), lambda i: (0, 0)),
            pl.BlockSpec((1, 11), lambda i: (0, 0)),
        ],
        out_specs=pl.BlockSpec((_BB, 11), lambda i: (i, 0)),
        out_shape=jax.ShapeDtypeStruct((BATCH, 11), jnp.float32),
    )(xu, xi, idu, idi, w1u, w1i, b1, w2, b2)


def kernel(user_ids, item_ids, user_table, item_table, W1, b1, W2, b2):
    uid = user_ids.astype(jnp.int32)
    iid = item_ids.astype(jnp.int32)
    eye = jnp.eye(PWIDTH, dtype=jnp.float32)
    uptab = _tc_repack(user_table.T, eye)
    urows = _sc_gather(uid & (GSIZE - 1), uptab)
    iptab = _tc_repack(item_table.T, eye)
    irows = _sc_gather(iid & (GSIZE - 1), iptab)
    return _tc_mlp(
        urows, irows,
        uid.reshape(BATCH, 1), iid.reshape(BATCH, 1),
        jnp.tile(W1[:EMBED_DIM], (PACK, 1)), jnp.tile(W1[EMBED_DIM:], (PACK, 1)),
        b1.reshape(1, 64), W2, b2.reshape(1, 11),
    )


# speedup vs baseline: 3.9878x; 1.1326x over previous
import functools

import jax
import jax.numpy as jnp
from jax import lax
from jax.experimental import pallas as pl
from jax.experimental.pallas import tpu as pltpu
from jax.experimental.pallas import tpu_sc as plsc

BATCH = 16384
EMBED_DIM = 32
NROWS = 1000000
PACK = 4
GSIZE = 1 << 18
PROWS = GSIZE
PWIDTH = EMBED_DIM * PACK

_INFO = plsc.get_sparse_core_info()
_NC = _INFO.num_cores
_NS = _INFO.num_subcores
_NW = _NC * _NS
_BPW = BATCH // _NW
_CHUNK = 128
_NCHUNK = _BPW // _CHUNK


_UBLK = 8192

_CONTRACT00 = (((0,), (0,)), ((), ()))


def _repack_body(t0_ref, t1_ref, t2_ref, t3_ref, eye_ref, o_ref):
    x = jnp.concatenate(
        [t0_ref[...], t1_ref[...], t2_ref[...], t3_ref[...]], axis=0)
    o_ref[...] = jax.lax.dot_general(x, eye_ref[...], _CONTRACT00,
                                     preferred_element_type=jnp.float32)


def _tc_repack(tab_t, eye):
    grid = (GSIZE // _UBLK,)
    gstride = GSIZE // _UBLK
    last_blk = NROWS // _UBLK

    def in_map(g):
        return lambda i: (0, jnp.minimum(g * gstride + i, last_blk))

    return pl.pallas_call(
        _repack_body,
        grid=grid,
        in_specs=(
            [pl.BlockSpec((EMBED_DIM, _UBLK), in_map(g)) for g in range(PACK)]
            + [pl.BlockSpec((PWIDTH, PWIDTH), lambda i: (0, 0))]
        ),
        out_specs=pl.BlockSpec((_UBLK, PWIDTH), lambda i: (i, 0)),
        out_shape=jax.ShapeDtypeStruct((PROWS, PWIDTH), jnp.float32),
    )(tab_t, tab_t, tab_t, tab_t, eye)


@functools.partial(
    pl.kernel,
    mesh=plsc.VectorSubcoreMesh(core_axis_name="c", subcore_axis_name="s"),
    out_type=jax.ShapeDtypeStruct((BATCH, PWIDTH), jnp.float32),
    scratch_types=[
        pltpu.VMEM((_BPW,), jnp.int32),
        pltpu.VMEM((_BPW, PWIDTH), jnp.float32),
        pltpu.SemaphoreType.DMA,
    ],
)
def _sc_gather(idx_hbm, ptab_hbm, out_hbm, idx_v, rows_v, sem):
    wid = lax.axis_index("s") * _NC + lax.axis_index("c")
    base = wid * _BPW
    pltpu.sync_copy(idx_hbm.at[pl.ds(base, _BPW)], idx_v)
    pltpu.async_copy(ptab_hbm.at[idx_v], rows_v, sem).wait()
    pltpu.sync_copy(rows_v, out_hbm.at[pl.ds(base, _BPW)])


_BB = 2048


def _mlp_body(xu_ref, xi_ref, idu_ref, idi_ref, w1u_ref, w1i_ref, b1_ref,
              w2_ref, b2_ref, o_ref):
    lane_g = jax.lax.broadcasted_iota(jnp.int32, (_BB, PWIDTH), 1) >> 5

    def masked(x128, ids):
        return jnp.where(lane_g == (ids >> 18), x128, 0.0)

    xu = masked(xu_ref[...], idu_ref[...])
    xi = masked(xi_ref[...], idi_ref[...])
    h = (
        jnp.dot(xu, w1u_ref[...], preferred_element_type=jnp.float32)
        + jnp.dot(xi, w1i_ref[...], preferred_element_type=jnp.float32)
        + b1_ref[...]
    )
    h = jnp.maximum(h, 0.0)
    o_ref[...] = (
        jnp.dot(h, w2_ref[...], preferred_element_type=jnp.float32) + b2_ref[...]
    )


def _tc_mlp(xu, xi, idu, idi, w1u, w1i, b1, w2, b2):
    grid = (BATCH // _BB,)
    return pl.pallas_call(
        _mlp_body,
        grid=grid,
        in_specs=[
            pl.BlockSpec((_BB, PWIDTH), lambda i: (i, 0)),
            pl.BlockSpec((_BB, PWIDTH), lambda i: (i, 0)),
            pl.BlockSpec((_BB, 1), lambda i: (i, 0)),
            pl.BlockSpec((_BB, 1), lambda i: (i, 0)),
            pl.BlockSpec((PWIDTH, 64), lambda i: (0, 0)),
            pl.BlockSpec((PWIDTH, 64), lambda i: (0, 0)),
            pl.BlockSpec((1, 64), lambda i: (0, 0)),
            pl.BlockSpec((64, 11), lambda i: (0, 0)),
            pl.BlockSpec((1, 11), lambda i: (0, 0)),
        ],
        out_specs=pl.BlockSpec((_BB, 11), lambda i: (i, 0)),
        out_shape=jax.ShapeDtypeStruct((BATCH, 11), jnp.float32),
    )(xu, xi, idu, idi, w1u, w1i, b1, w2, b2)


def kernel(user_ids, item_ids, user_table, item_table, W1, b1, W2, b2):
    uid = user_ids.astype(jnp.int32)
    iid = item_ids.astype(jnp.int32)
    eye = jnp.eye(PWIDTH, dtype=jnp.float32)
    uptab = _tc_repack(user_table.T, eye)
    urows = _sc_gather(uid & (GSIZE - 1), uptab)
    iptab = _tc_repack(item_table.T, eye)
    irows = _sc_gather(iid & (GSIZE - 1), iptab)
    return _tc_mlp(
        urows, irows,
        uid.reshape(BATCH, 1), iid.reshape(BATCH, 1),
        jnp.tile(W1[:EMBED_DIM], (PACK, 1)), jnp.tile(W1[EMBED_DIM:], (PACK, 1)),
        b1.reshape(1, 64), W2, b2.reshape(1, 11),
    )


# speedup vs baseline: 4.0668x; 1.0198x over previous
import functools

import jax
import jax.numpy as jnp
from jax import lax
from jax.experimental import pallas as pl
from jax.experimental.pallas import tpu as pltpu
from jax.experimental.pallas import tpu_sc as plsc

BATCH = 16384
EMBED_DIM = 32
NROWS = 1000000
PACK = 4
GSIZE = 1 << 18
PROWS = GSIZE
PWIDTH = EMBED_DIM * PACK

_INFO = plsc.get_sparse_core_info()
_NC = _INFO.num_cores
_NS = _INFO.num_subcores
_NW = _NC * _NS
_BPW = BATCH // _NW
_CHUNK = 128
_NCHUNK = _BPW // _CHUNK


_UBLK = 16384

_CONTRACT00 = (((0,), (0,)), ((), ()))


def _repack_body(t0_ref, t1_ref, t2_ref, t3_ref, eye_ref, o_ref):
    x = jnp.concatenate(
        [t0_ref[...], t1_ref[...], t2_ref[...], t3_ref[...]], axis=0)
    o_ref[...] = jax.lax.dot_general(x, eye_ref[...], _CONTRACT00,
                                     preferred_element_type=jnp.float32)


def _tc_repack(tab_t, eye):
    grid = (GSIZE // _UBLK,)
    gstride = GSIZE // _UBLK
    last_blk = NROWS // _UBLK

    def in_map(g):
        return lambda i: (0, jnp.minimum(g * gstride + i, last_blk))

    return pl.pallas_call(
        _repack_body,
        grid=grid,
        in_specs=(
            [pl.BlockSpec((EMBED_DIM, _UBLK), in_map(g)) for g in range(PACK)]
            + [pl.BlockSpec((PWIDTH, PWIDTH), lambda i: (0, 0))]
        ),
        out_specs=pl.BlockSpec((_UBLK, PWIDTH), lambda i: (i, 0)),
        out_shape=jax.ShapeDtypeStruct((PROWS, PWIDTH), jnp.float32),
    )(tab_t, tab_t, tab_t, tab_t, eye)


@functools.partial(
    pl.kernel,
    mesh=plsc.VectorSubcoreMesh(core_axis_name="c", subcore_axis_name="s"),
    out_type=jax.ShapeDtypeStruct((BATCH, PWIDTH), jnp.float32),
    scratch_types=[
        pltpu.VMEM((_BPW,), jnp.int32),
        pltpu.VMEM((_BPW, PWIDTH), jnp.float32),
        pltpu.SemaphoreType.DMA,
    ],
)
def _sc_gather(idx_hbm, ptab_hbm, out_hbm, idx_v, rows_v, sem):
    wid = lax.axis_index("s") * _NC + lax.axis_index("c")
    base = wid * _BPW
    pltpu.sync_copy(idx_hbm.at[pl.ds(base, _BPW)], idx_v)
    pltpu.async_copy(ptab_hbm.at[idx_v], rows_v, sem).wait()
    pltpu.sync_copy(rows_v, out_hbm.at[pl.ds(base, _BPW)])


_BB = 2048


def _mlp_body(xu_ref, xi_ref, idu_ref, idi_ref, w1u_ref, w1i_ref, b1_ref,
              w2_ref, b2_ref, o_ref):
    lane_g = jax.lax.broadcasted_iota(jnp.int32, (_BB, PWIDTH), 1) >> 5

    def masked(x128, ids):
        return jnp.where(lane_g == (ids >> 18), x128, 0.0)

    xu = masked(xu_ref[...], idu_ref[...])
    xi = masked(xi_ref[...], idi_ref[...])
    h = (
        jnp.dot(xu, w1u_ref[...], preferred_element_type=jnp.float32)
        + jnp.dot(xi, w1i_ref[...], preferred_element_type=jnp.float32)
        + b1_ref[...]
    )
    h = jnp.maximum(h, 0.0)
    o_ref[...] = (
        jnp.dot(h, w2_ref[...], preferred_element_type=jnp.float32) + b2_ref[...]
    )


def _tc_mlp(xu, xi, idu, idi, w1u, w1i, b1, w2, b2):
    grid = (BATCH // _BB,)
    return pl.pallas_call(
        _mlp_body,
        grid=grid,
        in_specs=[
            pl.BlockSpec((_BB, PWIDTH), lambda i: (i, 0)),
            pl.BlockSpec((_BB, PWIDTH), lambda i: (i, 0)),
            pl.BlockSpec((_BB, 1), lambda i: (i, 0)),
            pl.BlockSpec((_BB, 1), lambda i: (i, 0)),
            pl.BlockSpec((PWIDTH, 64), lambda i: (0, 0)),
            pl.BlockSpec((PWIDTH, 64), lambda i: (0, 0)),
            pl.BlockSpec((1, 64), lambda i: (0, 0)),
            pl.BlockSpec((64, 11), lambda i: (0, 0)),
            pl.BlockSpec((1, 11), lambda i: (0, 0)),
        ],
        out_specs=pl.BlockSpec((_BB, 11), lambda i: (i, 0)),
        out_shape=jax.ShapeDtypeStruct((BATCH, 11), jnp.float32),
    )(xu, xi, idu, idi, w1u, w1i, b1, w2, b2)


def kernel(user_ids, item_ids, user_table, item_table, W1, b1, W2, b2):
    uid = user_ids.astype(jnp.int32)
    iid = item_ids.astype(jnp.int32)
    eye = jnp.eye(PWIDTH, dtype=jnp.float32)
    uptab = _tc_repack(user_table.T, eye)
    urows = _sc_gather(uid & (GSIZE - 1), uptab)
    iptab = _tc_repack(item_table.T, eye)
    irows = _sc_gather(iid & (GSIZE - 1), iptab)
    return _tc_mlp(
        urows, irows,
        uid.reshape(BATCH, 1), iid.reshape(BATCH, 1),
        jnp.tile(W1[:EMBED_DIM], (PACK, 1)), jnp.tile(W1[EMBED_DIM:], (PACK, 1)),
        b1.reshape(1, 64), W2, b2.reshape(1, 11),
    )


# speedup vs baseline: 4.3127x; 1.0605x over previous
import functools

import jax
import jax.numpy as jnp
from jax import lax
from jax.experimental import pallas as pl
from jax.experimental.pallas import tpu as pltpu
from jax.experimental.pallas import tpu_sc as plsc

BATCH = 16384
EMBED_DIM = 32
NROWS = 1000000
PACK = 4
GSIZE = 1 << 18
PROWS = GSIZE
PWIDTH = EMBED_DIM * PACK

_INFO = plsc.get_sparse_core_info()
_NC = _INFO.num_cores
_NS = _INFO.num_subcores
_NW = _NC * _NS
_BPW = BATCH // _NW


_UBLK = 16384

_CONTRACT00 = (((0,), (0,)), ((), ()))


def _repack_body(t0_ref, t1_ref, t2_ref, t3_ref, eye_ref, o_ref):
    x = jnp.concatenate(
        [t0_ref[...], t1_ref[...], t2_ref[...], t3_ref[...]], axis=0)
    o_ref[...] = jax.lax.dot_general(x, eye_ref[...], _CONTRACT00,
                                     preferred_element_type=jnp.float32)


def _tc_repack(tab_t, eye):
    grid = (GSIZE // _UBLK,)
    gstride = GSIZE // _UBLK
    last_blk = NROWS // _UBLK

    def in_map(g):
        return lambda i: (0, jnp.minimum(g * gstride + i, last_blk))

    return pl.pallas_call(
        _repack_body,
        grid=grid,
        in_specs=(
            [pl.BlockSpec((EMBED_DIM, _UBLK), in_map(g)) for g in range(PACK)]
            + [pl.BlockSpec((PWIDTH, PWIDTH), lambda i: (0, 0))]
        ),
        out_specs=pl.BlockSpec((_UBLK, PWIDTH), lambda i: (i, 0)),
        out_shape=jax.ShapeDtypeStruct((PROWS, PWIDTH), jnp.float32),
    )(tab_t, tab_t, tab_t, tab_t, eye)


@functools.partial(
    pl.kernel,
    mesh=plsc.VectorSubcoreMesh(core_axis_name="c", subcore_axis_name="s"),
    out_type=jax.ShapeDtypeStruct((EMBED_DIM, BATCH), jnp.float32),
    scratch_types=[
        pltpu.VMEM((_BPW,), jnp.int32),
        pltpu.VMEM((_BPW,), jnp.int32),
        pltpu.VMEM((_BPW, PWIDTH), jnp.float32),
        pltpu.VMEM((EMBED_DIM, _BPW), jnp.float32),
        pltpu.SemaphoreType.DMA,
    ],
    compiler_params=pltpu.CompilerParams(needs_layout_passes=False),
)
def _sc_gather(ids_hbm, ptab_hbm, out_hbm, ids_v, row_v, rows_v, cols_v, sem):
    wid = lax.axis_index("s") * _NC + lax.axis_index("c")
    base = wid * _BPW
    pltpu.sync_copy(ids_hbm.at[pl.ds(base, _BPW)], ids_v)
    for j0 in range(0, _BPW, 16):
        row_v[pl.ds(j0, 16)] = ids_v[pl.ds(j0, 16)] & (GSIZE - 1)
    copy = pltpu.async_copy(ptab_hbm.at[row_v], rows_v, sem)
    iota16 = lax.iota(jnp.int32, 16)
    copy.wait()

    def extract(g, carry):
        j0 = g * 16
        grp = ids_v[pl.ds(j0, 16)] >> 18
        col0 = grp * EMBED_DIM
        ridx = iota16 + j0
        for c in range(EMBED_DIM):
            cols_v[c, pl.ds(j0, 16)] = plsc.load_gather(
                rows_v, [ridx, col0 + c])
        return carry

    lax.fori_loop(0, _BPW // 16, extract, 0)
    pltpu.sync_copy(cols_v, out_hbm.at[:, pl.ds(base, _BPW)])


_BB = 2048


def _mlp_body(xu_ref, xi_ref, w1u_ref, w1i_ref, b1_ref, w2_ref, b2t_ref,
              o_ref):
    h = (
        jax.lax.dot_general(xu_ref[...], w1u_ref[...], _CONTRACT00,
                            preferred_element_type=jnp.float32)
        + jax.lax.dot_general(xi_ref[...], w1i_ref[...], _CONTRACT00,
                              preferred_element_type=jnp.float32)
        + b1_ref[...]
    )
    h = jnp.maximum(h, 0.0)
    o_ref[...] = (
        jax.lax.dot_general(w2_ref[...], h, (((0,), (1,)), ((), ())),
                            preferred_element_type=jnp.float32)
        + b2t_ref[...]
    )


def _tc_mlp_t(xu_t, xi_t, w1u, w1i, b1, w2, b2t):
    grid = (BATCH // _BB,)
    return pl.pallas_call(
        _mlp_body,
        grid=grid,
        in_specs=[
            pl.BlockSpec((EMBED_DIM, _BB), lambda i: (0, i)),
            pl.BlockSpec((EMBED_DIM, _BB), lambda i: (0, i)),
            pl.BlockSpec((EMBED_DIM, 64), lambda i: (0, 0)),
            pl.BlockSpec((EMBED_DIM, 64), lambda i: (0, 0)),
            pl.BlockSpec((1, 64), lambda i: (0, 0)),
            pl.BlockSpec((64, 11), lambda i: (0, 0)),
            pl.BlockSpec((11, 1), lambda i: (0, 0)),
        ],
        out_specs=pl.BlockSpec((11, _BB), lambda i: (0, i)),
        out_shape=jax.ShapeDtypeStruct((11, BATCH), jnp.float32),
    )(xu_t, xi_t, w1u, w1i, b1, w2, b2t)


def kernel(user_ids, item_ids, user_table, item_table, W1, b1, W2, b2):
    uid = user_ids.astype(jnp.int32)
    iid = item_ids.astype(jnp.int32)
    eye = jnp.eye(PWIDTH, dtype=jnp.float32)
    uptab = _tc_repack(user_table.T, eye)
    urows_t = _sc_gather(uid, uptab)
    iptab = _tc_repack(item_table.T, eye)
    irows_t = _sc_gather(iid, iptab)
    out_t = _tc_mlp_t(
        urows_t, irows_t,
        W1[:EMBED_DIM], W1[EMBED_DIM:],
        b1.reshape(1, 64), W2, b2.reshape(11, 1),
    )
    return out_t.T


# speedup vs baseline: 4.3518x; 1.0091x over previous
import functools

import jax
import jax.numpy as jnp
from jax import lax
from jax.experimental import pallas as pl
from jax.experimental.pallas import tpu as pltpu
from jax.experimental.pallas import tpu_sc as plsc

BATCH = 16384
EMBED_DIM = 32
NROWS = 1000000
PACK = 4
GSIZE = 1 << 18
PROWS = GSIZE
PWIDTH = EMBED_DIM * PACK

_INFO = plsc.get_sparse_core_info()
_NC = _INFO.num_cores
_NS = _INFO.num_subcores
_NW = _NC * _NS
_BPW = BATCH // _NW


_UBLK = 16384

_CONTRACT00 = (((0,), (0,)), ((), ()))


def _repack_body(t0_ref, t1_ref, t2_ref, t3_ref, eye_ref, o_ref):
    x = jnp.concatenate(
        [t0_ref[...], t1_ref[...], t2_ref[...], t3_ref[...]], axis=0)
    o_ref[...] = jax.lax.dot_general(x, eye_ref[...], _CONTRACT00,
                                     preferred_element_type=jnp.float32)


def _tc_repack(tab_t, eye):
    grid = (GSIZE // _UBLK,)
    gstride = GSIZE // _UBLK
    last_blk = NROWS // _UBLK

    def in_map(g):
        return lambda i: (0, jnp.minimum(g * gstride + i, last_blk))

    return pl.pallas_call(
        _repack_body,
        grid=grid,
        in_specs=(
            [pl.BlockSpec((EMBED_DIM, _UBLK), in_map(g)) for g in range(PACK)]
            + [pl.BlockSpec((PWIDTH, PWIDTH), lambda i: (0, 0))]
        ),
        out_specs=pl.BlockSpec((_UBLK, PWIDTH), lambda i: (i, 0)),
        out_shape=jax.ShapeDtypeStruct((PROWS, PWIDTH), jnp.float32),
    )(tab_t, tab_t, tab_t, tab_t, eye)


@functools.partial(
    pl.kernel,
    mesh=plsc.VectorSubcoreMesh(core_axis_name="c", subcore_axis_name="s"),
    out_type=jax.ShapeDtypeStruct((EMBED_DIM, BATCH), jnp.float32),
    scratch_types=[
        pltpu.VMEM((_BPW,), jnp.int32),
        pltpu.VMEM((_BPW,), jnp.int32),
        pltpu.VMEM((_BPW, PWIDTH), jnp.float32),
        pltpu.VMEM((EMBED_DIM, _BPW), jnp.float32),
        pltpu.SemaphoreType.DMA,
    ],
    compiler_params=pltpu.CompilerParams(needs_layout_passes=False),
)
def _sc_gather(ids_hbm, ptab_hbm, out_hbm, ids_v, row_v, rows_v, cols_v, sem):
    wid = lax.axis_index("s") * _NC + lax.axis_index("c")
    base = wid * _BPW
    pltpu.sync_copy(ids_hbm.at[pl.ds(base, _BPW)], ids_v)
    for j0 in range(0, _BPW, 16):
        row_v[pl.ds(j0, 16)] = ids_v[pl.ds(j0, 16)] & (GSIZE - 1)
    copy = pltpu.async_copy(ptab_hbm.at[row_v], rows_v, sem)
    iota16 = lax.iota(jnp.int32, 16)
    copy.wait()

    def extract(g, carry):
        j0 = g * 16
        grp = ids_v[pl.ds(j0, 16)] >> 18
        col0 = grp * EMBED_DIM
        ridx = iota16 + j0
        for c in range(EMBED_DIM):
            cols_v[c, pl.ds(j0, 16)] = plsc.load_gather(
                rows_v, [ridx, col0 + c])
        return carry

    lax.fori_loop(0, _BPW // 16, extract, 0)
    pltpu.sync_copy(cols_v, out_hbm.at[:, pl.ds(base, _BPW)])


_BB = 4096


def _mlp_body(xu_ref, xi_ref, w1u_ref, w1i_ref, b1_ref, w2_ref, b2t_ref,
              o_ref):
    h = (
        jax.lax.dot_general(xu_ref[...], w1u_ref[...], _CONTRACT00,
                            preferred_element_type=jnp.float32)
        + jax.lax.dot_general(xi_ref[...], w1i_ref[...], _CONTRACT00,
                              preferred_element_type=jnp.float32)
        + b1_ref[...]
    )
    h = jnp.maximum(h, 0.0)
    o_ref[...] = (
        jax.lax.dot_general(w2_ref[...], h, (((0,), (1,)), ((), ())),
                            preferred_element_type=jnp.float32)
        + b2t_ref[...]
    )


def _tc_mlp_t(xu_t, xi_t, w1u, w1i, b1, w2, b2t):
    grid = (BATCH // _BB,)
    return pl.pallas_call(
        _mlp_body,
        grid=grid,
        in_specs=[
            pl.BlockSpec((EMBED_DIM, _BB), lambda i: (0, i)),
            pl.BlockSpec((EMBED_DIM, _BB), lambda i: (0, i)),
            pl.BlockSpec((EMBED_DIM, 64), lambda i: (0, 0)),
            pl.BlockSpec((EMBED_DIM, 64), lambda i: (0, 0)),
            pl.BlockSpec((1, 64), lambda i: (0, 0)),
            pl.BlockSpec((64, 11), lambda i: (0, 0)),
            pl.BlockSpec((11, 1), lambda i: (0, 0)),
        ],
        out_specs=pl.BlockSpec((11, _BB), lambda i: (0, i)),
        out_shape=jax.ShapeDtypeStruct((11, BATCH), jnp.float32),
    )(xu_t, xi_t, w1u, w1i, b1, w2, b2t)


def kernel(user_ids, item_ids, user_table, item_table, W1, b1, W2, b2):
    uid = user_ids.astype(jnp.int32)
    iid = item_ids.astype(jnp.int32)
    eye = jnp.eye(PWIDTH, dtype=jnp.float32)
    uptab = _tc_repack(user_table.T, eye)
    urows_t = _sc_gather(uid, uptab)
    iptab = _tc_repack(item_table.T, eye)
    irows_t = _sc_gather(iid, iptab)
    out_t = _tc_mlp_t(
        urows_t, irows_t,
        W1[:EMBED_DIM], W1[EMBED_DIM:],
        b1.reshape(1, 64), W2, b2.reshape(11, 1),
    )
    return out_t.T
